# Initial kernel scaffold; baseline (speedup 1.0000x reference)
import jax
import jax.numpy as jnp
from jax.experimental import pallas as pl


def kernel(x, y, senders, receivers, rel_pos, window_support, a, S1, W1, b1, g1, o1, S2, W2, b2, g2, o2, S3, W3, b3, g3, o3, S4, W4, b4, g4, o4):
    raise NotImplementedError("write your pallas kernel here")


# speedup vs baseline: 110.2011x; 110.2011x over previous
import functools

import jax
import jax.numpy as jnp
from jax import lax
from jax.experimental import pallas as pl
from jax.experimental.pallas import tpu as pltpu
from jax.experimental.pallas import tpu_sc as plsc

N = 10000
E = 640000
LANES = 128
NC, NS = 2, 16
NW = NC * NS
RCH = 8
NCH = 20
RPW = RCH * NCH
ROWS = NW * RPW
EPAD = ROWS * LANES
ACCN = 10240
NPS = ACCN // NS
CH_E = RCH * LANES


def _edge_w_body(rpx_ref, rpy_ref, win_ref, sa_ref, o1_ref, o2_ref, o3_ref,
                 o4_ref):
    rx = rpx_ref[...]
    ry = rpy_ref[...]
    win = win_ref[...]
    ua = (jnp.clip(rx, -1.0, 1.0) + 1.0) * 1.5
    ub = (jnp.clip(ry, -1.0, 1.0) + 1.0) * 1.5
    ia = jnp.clip(jnp.floor(ua), 0.0, 2.0)
    ib = jnp.clip(jnp.floor(ub), 0.0, 2.0)
    fa = ua - ia
    fb = ub - ib
    c00 = (1.0 - fa) * (1.0 - fb)
    c01 = (1.0 - fa) * fb
    c10 = fa * (1.0 - fb)
    c11 = fa * fb
    ea = [(ia == float(k)).astype(jnp.float32) for k in range(3)]
    eb = [(ib == float(m)).astype(jnp.float32) for m in range(3)]
    outs = [o1_ref, o2_ref, o3_ref, o4_ref]
    for l in range(4):
        acc = None
        for k in range(3):
            for m in range(3):
                j = k * 3 + m
                br = (c00 * sa_ref[l * 36 + j]
                      + c01 * sa_ref[l * 36 + 9 + j]
                      + c10 * sa_ref[l * 36 + 18 + j]
                      + c11 * sa_ref[l * 36 + 27 + j])
                term = (ea[k] * eb[m]) * br
                acc = term if acc is None else acc + term
        outs[l][...] = acc * win


_BR = 64

_edge_w = pl.pallas_call(
    _edge_w_body,
    grid=(ROWS // _BR,),
    in_specs=[pl.BlockSpec((_BR, LANES), lambda i: (i, 0))] * 3
    + [pl.BlockSpec(memory_space=pltpu.SMEM)],
    out_specs=[pl.BlockSpec((_BR, LANES), lambda i: (i, 0))] * 4,
    out_shape=[jax.ShapeDtypeStruct((ROWS, LANES), jnp.float32)] * 4,
)


def _corner_pack(S):
    return jnp.stack(
        [S[0:3, 0:3], S[0:3, 1:4], S[1:4, 0:3], S[1:4, 1:4]]).reshape(36)


@functools.lru_cache(maxsize=None)
def _make_conv(F):
    mesh = plsc.VectorSubcoreMesh(core_axis_name="c", subcore_axis_name="s",
                                  num_cores=NC, num_subcores=NS)

    def body(feat, send, recv, wgt, out, sidx, ridx, wv, rows, acc, sem):
        cid = lax.axis_index("c")
        sid = lax.axis_index("s")
        wid = cid * NS + sid

        def zrow(r, carry):
            for q in range(F // 16):
                rows[r, pl.ds(q * 16, 16)] = jnp.zeros((16,), jnp.float32)
            return carry

        lax.fori_loop(0, NPS, zrow, 0)
        pltpu.sync_copy(rows.at[pl.ds(0, NPS)], acc.at[pl.ds(sid * NPS, NPS)])
        plsc.subcore_barrier()

        def chunk(ch, carry):
            row0 = wid * RPW + ch * RCH
            pltpu.sync_copy(send.at[pl.ds(row0, RCH)], sidx)
            pltpu.sync_copy(recv.at[pl.ds(row0, RCH)], ridx)
            pltpu.sync_copy(wgt.at[pl.ds(row0, RCH)], wv)
            descs = [
                pltpu.async_copy(feat.at[sidx.at[j]],
                                 rows.at[pl.ds(j * LANES, LANES)], sem)
                for j in range(RCH)
            ]
            for d in descs:
                d.wait()

            for j in range(RCH):
                def scale_grp(i, carry2, j=j):
                    w16 = wv[j, pl.ds(i * 16, 16)]
                    for l2 in range(16):
                        w_s = w16[l2]
                        e = j * LANES + i * 16 + l2
                        for q in range(F // 16):
                            rows[e, pl.ds(q * 16, 16)] = (
                                rows[e, pl.ds(q * 16, 16)] * w_s)
                    return carry2

                lax.fori_loop(0, LANES // 16, scale_grp, 0)

            for j in range(RCH):
                pltpu.sync_copy(rows.at[pl.ds(j * LANES, LANES)],
                                acc.at[ridx.at[j]], add=True)
            return carry

        lax.fori_loop(0, NCH, chunk, 0)
        plsc.subcore_barrier()

        pltpu.sync_copy(acc.at[pl.ds(sid * NPS, NPS)], rows.at[pl.ds(0, NPS)])
        pltpu.sync_copy(rows.at[pl.ds(0, NPS)],
                        out.at[pl.ds(cid * ACCN + sid * NPS, NPS)])

    return pl.kernel(
        body,
        out_type=jax.ShapeDtypeStruct((NC * ACCN, F), jnp.float32),
        mesh=mesh,
        compiler_params=pltpu.CompilerParams(use_tc_tiling_on_sc=False),
        scratch_types=[
            pltpu.VMEM((RCH, LANES), jnp.int32),
            pltpu.VMEM((RCH, LANES), jnp.int32),
            pltpu.VMEM((RCH, LANES), jnp.float32),
            pltpu.VMEM((CH_E, F), jnp.float32),
            pltpu.VMEM_SHARED((ACCN, F), jnp.float32),
            pltpu.SemaphoreType.DMA,
        ],
    )


def _dense_mid_body(p_ref, a_ref, w_ref, b_ref, g_ref, o_ref, out_ref):
    agg = (p_ref[0:N, :] + p_ref[ACCN:ACCN + N, :]) / a_ref[...]
    t = jnp.dot(agg, w_ref[...], preferred_element_type=jnp.float32) + b_ref[...]
    m = jnp.mean(t, axis=0, keepdims=True)
    v = jnp.mean((t - m) * (t - m), axis=0, keepdims=True)
    h = g_ref[...] * (t - m) * lax.rsqrt(v + 1e-5) + o_ref[...]
    out_ref[...] = jnp.maximum(h, 0.0)


def _dense_gate_body(p_ref, a_ref, w_ref, b_ref, g_ref, o_ref, x_ref, y_ref,
                     out_ref):
    agg = (p_ref[0:N, :] + p_ref[ACCN:ACCN + N, :]) / a_ref[...]
    t = jnp.dot(agg, w_ref[...], preferred_element_type=jnp.float32) + b_ref[...]
    m = jnp.mean(t, axis=0, keepdims=True)
    v = jnp.mean((t - m) * (t - m), axis=0, keepdims=True)
    h = g_ref[...] * (t - m) * lax.rsqrt(v + 1e-5) + o_ref[...]
    wei = jax.nn.sigmoid(h)
    out_ref[...] = 2.0 * x_ref[...] * wei + 2.0 * y_ref[...] * (1.0 - wei)


def _make_dense_mid(fin, fout):
    return pl.pallas_call(
        _dense_mid_body,
        out_shape=jax.ShapeDtypeStruct((N, fout), jnp.float32),
    )


def _make_dense_gate(fin, fout):
    return pl.pallas_call(
        _dense_gate_body,
        out_shape=jax.ShapeDtypeStruct((N, fout), jnp.float32),
    )


_dense1 = _make_dense_mid(64, 64)
_dense2 = _make_dense_gate(64, 32)
_dense3 = _make_dense_mid(32, 64)
_dense4 = _make_dense_gate(64, 32)


def kernel(x, y, senders, receivers, rel_pos, window_support, a,
           S1, W1, b1, g1, o1, S2, W2, b2, g2, o2,
           S3, W3, b3, g3, o3, S4, W4, b4, g4, o4):
    f32 = jnp.float32
    npad = EPAD - E
    pad_i = (jnp.arange(npad, dtype=jnp.int32) % N)
    send2 = jnp.concatenate([senders.astype(jnp.int32), pad_i]).reshape(ROWS, LANES)
    recv2 = jnp.concatenate([receivers.astype(jnp.int32), pad_i]).reshape(ROWS, LANES)
    zpad = jnp.zeros((npad,), f32)
    rpx = jnp.concatenate([rel_pos[:, 0].astype(f32), zpad]).reshape(ROWS, LANES)
    rpy = jnp.concatenate([rel_pos[:, 1].astype(f32), zpad]).reshape(ROWS, LANES)
    win = jnp.concatenate([window_support.astype(f32), zpad]).reshape(ROWS, LANES)
    sa = jnp.concatenate(
        [_corner_pack(S) for S in (S1, S2, S3, S4)]).astype(f32)

    w1, w2, w3, w4 = _edge_w(rpx, rpy, win, sa)

    a2 = a.astype(f32).reshape(N, 1)
    xa = jnp.concatenate([x, y], axis=1).astype(f32)

    conv64 = _make_conv(64)
    conv32 = _make_conv(32)
    p1 = conv64(xa, send2, recv2, w1)
    h1 = _dense1(p1, a2, W1, b1.reshape(1, -1), g1.reshape(1, -1),
                 o1.reshape(1, -1))
    p2 = conv64(h1, send2, recv2, w2)
    xo = _dense2(p2, a2, W2, b2.reshape(1, -1), g2.reshape(1, -1),
                 o2.reshape(1, -1), x, y)
    p3 = conv32(xo, send2, recv2, w3)
    h3 = _dense3(p3, a2, W3, b3.reshape(1, -1), g3.reshape(1, -1),
                 o3.reshape(1, -1))
    p4 = conv64(h3, send2, recv2, w4)
    return _dense4(p4, a2, W4, b4.reshape(1, -1), g4.reshape(1, -1),
                   o4.reshape(1, -1), x, y)


# speedup vs baseline: 122.2813x; 1.1096x over previous
import functools

import jax
import jax.numpy as jnp
from jax import lax
from jax.experimental import pallas as pl
from jax.experimental.pallas import tpu as pltpu
from jax.experimental.pallas import tpu_sc as plsc

N = 10000
E = 640000
LANES = 128
NC, NS = 2, 16
NW = NC * NS
RCH = 8
NCH = 20
RPW = RCH * NCH
ROWS = NW * RPW
EPAD = ROWS * LANES
ACCN = 10240
NPS = ACCN // NS
CH_E = RCH * LANES


def _edge_w_body(rpx_ref, rpy_ref, win_ref, sa_ref, o1_ref, o2_ref, o3_ref,
                 o4_ref):
    rx = rpx_ref[...]
    ry = rpy_ref[...]
    win = win_ref[...]
    ua = (jnp.clip(rx, -1.0, 1.0) + 1.0) * 1.5
    ub = (jnp.clip(ry, -1.0, 1.0) + 1.0) * 1.5
    ia = jnp.clip(jnp.floor(ua), 0.0, 2.0)
    ib = jnp.clip(jnp.floor(ub), 0.0, 2.0)
    fa = ua - ia
    fb = ub - ib
    c00 = (1.0 - fa) * (1.0 - fb)
    c01 = (1.0 - fa) * fb
    c10 = fa * (1.0 - fb)
    c11 = fa * fb
    ea = [(ia == float(k)).astype(jnp.float32) for k in range(3)]
    eb = [(ib == float(m)).astype(jnp.float32) for m in range(3)]
    outs = [o1_ref, o2_ref, o3_ref, o4_ref]
    for l in range(4):
        acc = None
        for k in range(3):
            for m in range(3):
                j = k * 3 + m
                br = (c00 * sa_ref[l * 36 + j]
                      + c01 * sa_ref[l * 36 + 9 + j]
                      + c10 * sa_ref[l * 36 + 18 + j]
                      + c11 * sa_ref[l * 36 + 27 + j])
                term = (ea[k] * eb[m]) * br
                acc = term if acc is None else acc + term
        outs[l][...] = acc * win


_BR = 64

_edge_w = pl.pallas_call(
    _edge_w_body,
    grid=(ROWS // _BR,),
    in_specs=[pl.BlockSpec((_BR, LANES), lambda i: (i, 0))] * 3
    + [pl.BlockSpec(memory_space=pltpu.SMEM)],
    out_specs=[pl.BlockSpec((_BR, LANES), lambda i: (i, 0))] * 4,
    out_shape=[jax.ShapeDtypeStruct((ROWS, LANES), jnp.float32)] * 4,
)


def _corner_pack(S):
    return jnp.stack(
        [S[0:3, 0:3], S[0:3, 1:4], S[1:4, 0:3], S[1:4, 1:4]]).reshape(36)


@functools.lru_cache(maxsize=None)
def _make_conv(F):
    mesh = plsc.VectorSubcoreMesh(core_axis_name="c", subcore_axis_name="s",
                                  num_cores=NC, num_subcores=NS)

    HC = 4
    HCE = HC * LANES
    NHC = RPW // HC
    GRP = 2 * HC

    def body(feat, send, recv, wgt, out, sidx, ridx, wv, rows, acc, gsem,
             ssem):
        cid = lax.axis_index("c")
        sid = lax.axis_index("s")
        wid = cid * NS + sid

        def zrow(r, carry):
            for q in range(F // 16):
                rows[0, r, pl.ds(q * 16, 16)] = jnp.zeros((16,), jnp.float32)
                rows[1, r, pl.ds(q * 16, 16)] = jnp.zeros((16,), jnp.float32)
            return carry

        lax.fori_loop(0, HCE, zrow, 0)
        pltpu.sync_copy(rows.at[0], acc.at[pl.ds(sid * NPS, HCE)])
        pltpu.sync_copy(rows.at[1, pl.ds(0, NPS - HCE)],
                        acc.at[pl.ds(sid * NPS + HCE, NPS - HCE)])
        plsc.subcore_barrier()

        def load_grp(g):
            row0 = wid * RPW + g * GRP
            slot = g & 1
            pltpu.sync_copy(send.at[pl.ds(row0, GRP)], sidx.at[slot])
            pltpu.sync_copy(recv.at[pl.ds(row0, GRP)], ridx.at[slot])
            pltpu.sync_copy(wgt.at[pl.ds(row0, GRP)], wv.at[slot])

        def fire_gathers(hc):
            p = hc & 1
            slot = (hc >> 1) & 1
            h = (hc & 1) * HC
            for j in range(HC):
                pltpu.async_copy(feat.at[sidx.at[slot, h + j]],
                                 rows.at[p, pl.ds(j * LANES, LANES)], gsem)

        def drain_gathers(hc):
            p = hc & 1
            slot = (hc >> 1) & 1
            h = (hc & 1) * HC
            for j in range(HC):
                pltpu.make_async_copy(
                    feat.at[sidx.at[slot, h + j]],
                    rows.at[p, pl.ds(j * LANES, LANES)], gsem).wait()

        def fire_scatters(hc):
            p = hc & 1
            slot = (hc >> 1) & 1
            h = (hc & 1) * HC
            for j in range(HC):
                pltpu.async_copy(rows.at[p, pl.ds(j * LANES, LANES)],
                                 acc.at[ridx.at[slot, h + j]], ssem, add=True)

        def drain_scatters(hc):
            p = hc & 1
            slot = (hc >> 1) & 1
            h = (hc & 1) * HC
            for j in range(HC):
                pltpu.make_async_copy(
                    rows.at[p, pl.ds(j * LANES, LANES)],
                    acc.at[ridx.at[slot, h + j]], ssem).wait()

        load_grp(jnp.int32(0))
        fire_gathers(jnp.int32(0))

        def step(hc, carry):
            p = hc & 1
            slot = (hc >> 1) & 1
            h = (hc & 1) * HC
            drain_gathers(hc)

            @pl.when((hc + 1 < NHC) & (((hc + 1) & 1) == 0))
            def _():
                load_grp((hc + 1) >> 1)

            @pl.when(hc >= 1)
            def _():
                drain_scatters(hc - 1)

            @pl.when(hc + 1 < NHC)
            def _():
                fire_gathers(hc + 1)

            for j in range(HC):
                def scale_grp(i, carry2, j=j):
                    w16 = wv[slot, h + j, pl.ds(i * 16, 16)]
                    for l2 in range(16):
                        w_s = w16[l2]
                        e = j * LANES + i * 16 + l2
                        for q in range(F // 16):
                            rows[p, e, pl.ds(q * 16, 16)] = (
                                rows[p, e, pl.ds(q * 16, 16)] * w_s)
                    return carry2

                lax.fori_loop(0, LANES // 16, scale_grp, 0)

            fire_scatters(hc)
            return carry

        lax.fori_loop(0, NHC, step, 0)
        drain_scatters(jnp.int32(NHC - 1))
        plsc.subcore_barrier()

        pltpu.sync_copy(acc.at[pl.ds(sid * NPS, HCE)], rows.at[0])
        pltpu.sync_copy(rows.at[0],
                        out.at[pl.ds(cid * ACCN + sid * NPS, HCE)])
        pltpu.sync_copy(acc.at[pl.ds(sid * NPS + HCE, NPS - HCE)],
                        rows.at[1, pl.ds(0, NPS - HCE)])
        pltpu.sync_copy(rows.at[1, pl.ds(0, NPS - HCE)],
                        out.at[pl.ds(cid * ACCN + sid * NPS + HCE,
                                     NPS - HCE)])

    return pl.kernel(
        body,
        out_type=jax.ShapeDtypeStruct((NC * ACCN, F), jnp.float32),
        mesh=mesh,
        compiler_params=pltpu.CompilerParams(use_tc_tiling_on_sc=False),
        scratch_types=[
            pltpu.VMEM((2, GRP, LANES), jnp.int32),
            pltpu.VMEM((2, GRP, LANES), jnp.int32),
            pltpu.VMEM((2, GRP, LANES), jnp.float32),
            pltpu.VMEM((2, HCE, F), jnp.float32),
            pltpu.VMEM_SHARED((ACCN, F), jnp.float32),
            pltpu.SemaphoreType.DMA,
            pltpu.SemaphoreType.DMA,
        ],
    )


def _dense_mid_body(p_ref, a_ref, w_ref, b_ref, g_ref, o_ref, out_ref):
    agg = (p_ref[0:N, :] + p_ref[ACCN:ACCN + N, :]) / a_ref[...]
    t = jnp.dot(agg, w_ref[...], preferred_element_type=jnp.float32) + b_ref[...]
    m = jnp.mean(t, axis=0, keepdims=True)
    v = jnp.mean((t - m) * (t - m), axis=0, keepdims=True)
    h = g_ref[...] * (t - m) * lax.rsqrt(v + 1e-5) + o_ref[...]
    out_ref[...] = jnp.maximum(h, 0.0)


def _dense_gate_body(p_ref, a_ref, w_ref, b_ref, g_ref, o_ref, x_ref, y_ref,
                     out_ref):
    agg = (p_ref[0:N, :] + p_ref[ACCN:ACCN + N, :]) / a_ref[...]
    t = jnp.dot(agg, w_ref[...], preferred_element_type=jnp.float32) + b_ref[...]
    m = jnp.mean(t, axis=0, keepdims=True)
    v = jnp.mean((t - m) * (t - m), axis=0, keepdims=True)
    h = g_ref[...] * (t - m) * lax.rsqrt(v + 1e-5) + o_ref[...]
    wei = jax.nn.sigmoid(h)
    out_ref[...] = 2.0 * x_ref[...] * wei + 2.0 * y_ref[...] * (1.0 - wei)


def _make_dense_mid(fin, fout):
    return pl.pallas_call(
        _dense_mid_body,
        out_shape=jax.ShapeDtypeStruct((N, fout), jnp.float32),
    )


def _make_dense_gate(fin, fout):
    return pl.pallas_call(
        _dense_gate_body,
        out_shape=jax.ShapeDtypeStruct((N, fout), jnp.float32),
    )


_dense1 = _make_dense_mid(64, 64)
_dense2 = _make_dense_gate(64, 32)
_dense3 = _make_dense_mid(32, 64)
_dense4 = _make_dense_gate(64, 32)


def kernel(x, y, senders, receivers, rel_pos, window_support, a,
           S1, W1, b1, g1, o1, S2, W2, b2, g2, o2,
           S3, W3, b3, g3, o3, S4, W4, b4, g4, o4):
    f32 = jnp.float32
    npad = EPAD - E
    pad_i = (jnp.arange(npad, dtype=jnp.int32) % N)
    send2 = jnp.concatenate([senders.astype(jnp.int32), pad_i]).reshape(ROWS, LANES)
    recv2 = jnp.concatenate([receivers.astype(jnp.int32), pad_i]).reshape(ROWS, LANES)
    zpad = jnp.zeros((npad,), f32)
    rpx = jnp.concatenate([rel_pos[:, 0].astype(f32), zpad]).reshape(ROWS, LANES)
    rpy = jnp.concatenate([rel_pos[:, 1].astype(f32), zpad]).reshape(ROWS, LANES)
    win = jnp.concatenate([window_support.astype(f32), zpad]).reshape(ROWS, LANES)
    sa = jnp.concatenate(
        [_corner_pack(S) for S in (S1, S2, S3, S4)]).astype(f32)

    w1, w2, w3, w4 = _edge_w(rpx, rpy, win, sa)

    a2 = a.astype(f32).reshape(N, 1)
    xa = jnp.concatenate([x, y], axis=1).astype(f32)

    conv64 = _make_conv(64)
    conv32 = _make_conv(32)
    p1 = conv64(xa, send2, recv2, w1)
    h1 = _dense1(p1, a2, W1, b1.reshape(1, -1), g1.reshape(1, -1),
                 o1.reshape(1, -1))
    p2 = conv64(h1, send2, recv2, w2)
    xo = _dense2(p2, a2, W2, b2.reshape(1, -1), g2.reshape(1, -1),
                 o2.reshape(1, -1), x, y)
    p3 = conv32(xo, send2, recv2, w3)
    h3 = _dense3(p3, a2, W3, b3.reshape(1, -1), g3.reshape(1, -1),
                 o3.reshape(1, -1))
    p4 = conv64(h3, send2, recv2, w4)
    return _dense4(p4, a2, W4, b4.reshape(1, -1), g4.reshape(1, -1),
                   o4.reshape(1, -1), x, y)


# speedup vs baseline: 122.9330x; 1.0053x over previous
import functools

import jax
import jax.numpy as jnp
from jax import lax
from jax.experimental import pallas as pl
from jax.experimental.pallas import tpu as pltpu
from jax.experimental.pallas import tpu_sc as plsc

N = 10000
E = 640000
LANES = 128
NC, NS = 2, 16
NW = NC * NS
RCH = 8
NCH = 20
RPW = RCH * NCH
ROWS = NW * RPW
EPAD = ROWS * LANES
ACCN = 10240
NPS = ACCN // NS
CH_E = RCH * LANES


def _edge_w_body(rpx_ref, rpy_ref, win_ref, sa_ref, o1_ref, o2_ref, o3_ref,
                 o4_ref):
    rx = rpx_ref[...]
    ry = rpy_ref[...]
    win = win_ref[...]
    ua = (jnp.clip(rx, -1.0, 1.0) + 1.0) * 1.5
    ub = (jnp.clip(ry, -1.0, 1.0) + 1.0) * 1.5
    ia = jnp.clip(jnp.floor(ua), 0.0, 2.0)
    ib = jnp.clip(jnp.floor(ub), 0.0, 2.0)
    fa = ua - ia
    fb = ub - ib
    c00 = (1.0 - fa) * (1.0 - fb)
    c01 = (1.0 - fa) * fb
    c10 = fa * (1.0 - fb)
    c11 = fa * fb
    ea = [(ia == float(k)).astype(jnp.float32) for k in range(3)]
    eb = [(ib == float(m)).astype(jnp.float32) for m in range(3)]
    outs = [o1_ref, o2_ref, o3_ref, o4_ref]
    for l in range(4):
        acc = None
        for k in range(3):
            for m in range(3):
                j = k * 3 + m
                br = (c00 * sa_ref[l * 36 + j]
                      + c01 * sa_ref[l * 36 + 9 + j]
                      + c10 * sa_ref[l * 36 + 18 + j]
                      + c11 * sa_ref[l * 36 + 27 + j])
                term = (ea[k] * eb[m]) * br
                acc = term if acc is None else acc + term
        outs[l][...] = acc * win


_BR = 64

_edge_w = pl.pallas_call(
    _edge_w_body,
    grid=(ROWS // _BR,),
    in_specs=[pl.BlockSpec((_BR, LANES), lambda i: (i, 0))] * 3
    + [pl.BlockSpec(memory_space=pltpu.SMEM)],
    out_specs=[pl.BlockSpec((_BR, LANES), lambda i: (i, 0))] * 4,
    out_shape=[jax.ShapeDtypeStruct((ROWS, LANES), jnp.float32)] * 4,
)


def _corner_pack(S):
    return jnp.stack(
        [S[0:3, 0:3], S[0:3, 1:4], S[1:4, 0:3], S[1:4, 1:4]]).reshape(36)


@functools.lru_cache(maxsize=None)
def _make_conv(F):
    mesh = plsc.VectorSubcoreMesh(core_axis_name="c", subcore_axis_name="s",
                                  num_cores=NC, num_subcores=NS)

    HC = 4
    HCE = HC * LANES
    NPASS = 4
    PRW = RPW // NPASS
    NHC = PRW // HC

    def body(feat, send, recv, wgt, out, sidx, ridx, wv, rows, acc, gsem,
             ssem):
        cid = lax.axis_index("c")
        sid = lax.axis_index("s")
        wid = cid * NS + sid

        def zrow(r, carry):
            for q in range(F // 16):
                rows[0, r, pl.ds(q * 16, 16)] = jnp.zeros((16,), jnp.float32)
                rows[1, r, pl.ds(q * 16, 16)] = jnp.zeros((16,), jnp.float32)
            return carry

        lax.fori_loop(0, HCE, zrow, 0)
        pltpu.sync_copy(rows.at[0], acc.at[pl.ds(sid * NPS, HCE)])
        pltpu.sync_copy(rows.at[1, pl.ds(0, NPS - HCE)],
                        acc.at[pl.ds(sid * NPS + HCE, NPS - HCE)])
        plsc.subcore_barrier()

        def fire_gathers(hc):
            p = hc & 1
            for j in range(HC):
                pltpu.async_copy(feat.at[sidx.at[hc * HC + j]],
                                 rows.at[p, pl.ds(j * LANES, LANES)], gsem)

        def drain_gathers(hc):
            p = hc & 1
            for j in range(HC):
                pltpu.make_async_copy(
                    feat.at[sidx.at[hc * HC + j]],
                    rows.at[p, pl.ds(j * LANES, LANES)], gsem).wait()

        def fire_scatters(hc):
            p = hc & 1
            for j in range(HC):
                pltpu.async_copy(rows.at[p, pl.ds(j * LANES, LANES)],
                                 acc.at[ridx.at[hc * HC + j]], ssem, add=True)

        def drain_scatters(hc):
            p = hc & 1
            for j in range(HC):
                pltpu.make_async_copy(
                    rows.at[p, pl.ds(j * LANES, LANES)],
                    acc.at[ridx.at[hc * HC + j]], ssem).wait()

        def pass_body(pass_, pcarry):
            row0 = wid * RPW + pass_ * PRW
            pltpu.sync_copy(send.at[pl.ds(row0, PRW)], sidx)
            pltpu.sync_copy(recv.at[pl.ds(row0, PRW)], ridx)
            pltpu.sync_copy(wgt.at[pl.ds(row0, PRW)], wv)

            fire_gathers(jnp.int32(0))

            def step(hc, carry):
                p = hc & 1
                drain_gathers(hc)

                @pl.when(hc >= 1)
                def _():
                    drain_scatters(hc - 1)

                @pl.when(hc + 1 < NHC)
                def _():
                    fire_gathers(hc + 1)

                for j in range(HC):
                    @plsc.parallel_loop(0, LANES // 16, step=1, unroll=2)
                    def scale_grp(i, j=j):
                        w16 = wv[hc * HC + j, pl.ds(i * 16, 16)]
                        for l2 in range(16):
                            w_s = w16[l2]
                            e = j * LANES + i * 16 + l2
                            for q in range(F // 16):
                                rows[p, e, pl.ds(q * 16, 16)] = (
                                    rows[p, e, pl.ds(q * 16, 16)] * w_s)

                fire_scatters(hc)
                return carry

            lax.fori_loop(0, NHC, step, 0)
            drain_scatters(jnp.int32(NHC - 1))
            return pcarry

        lax.fori_loop(0, NPASS, pass_body, 0)
        plsc.subcore_barrier()

        pltpu.sync_copy(acc.at[pl.ds(sid * NPS, HCE)], rows.at[0])
        pltpu.sync_copy(rows.at[0],
                        out.at[pl.ds(cid * ACCN + sid * NPS, HCE)])
        pltpu.sync_copy(acc.at[pl.ds(sid * NPS + HCE, NPS - HCE)],
                        rows.at[1, pl.ds(0, NPS - HCE)])
        pltpu.sync_copy(rows.at[1, pl.ds(0, NPS - HCE)],
                        out.at[pl.ds(cid * ACCN + sid * NPS + HCE,
                                     NPS - HCE)])

    return pl.kernel(
        body,
        out_type=jax.ShapeDtypeStruct((NC * ACCN, F), jnp.float32),
        mesh=mesh,
        compiler_params=pltpu.CompilerParams(use_tc_tiling_on_sc=False),
        scratch_types=[
            pltpu.VMEM((PRW, LANES), jnp.int32),
            pltpu.VMEM((PRW, LANES), jnp.int32),
            pltpu.VMEM((PRW, LANES), jnp.float32),
            pltpu.VMEM((2, HCE, F), jnp.float32),
            pltpu.VMEM_SHARED((ACCN, F), jnp.float32),
            pltpu.SemaphoreType.DMA,
            pltpu.SemaphoreType.DMA,
        ],
    )


def _dense_mid_body(p_ref, a_ref, w_ref, b_ref, g_ref, o_ref, out_ref):
    agg = (p_ref[0:N, :] + p_ref[ACCN:ACCN + N, :]) / a_ref[...]
    t = jnp.dot(agg, w_ref[...], preferred_element_type=jnp.float32) + b_ref[...]
    m = jnp.mean(t, axis=0, keepdims=True)
    v = jnp.mean((t - m) * (t - m), axis=0, keepdims=True)
    h = g_ref[...] * (t - m) * lax.rsqrt(v + 1e-5) + o_ref[...]
    out_ref[...] = jnp.maximum(h, 0.0)


def _dense_gate_body(p_ref, a_ref, w_ref, b_ref, g_ref, o_ref, x_ref, y_ref,
                     out_ref):
    agg = (p_ref[0:N, :] + p_ref[ACCN:ACCN + N, :]) / a_ref[...]
    t = jnp.dot(agg, w_ref[...], preferred_element_type=jnp.float32) + b_ref[...]
    m = jnp.mean(t, axis=0, keepdims=True)
    v = jnp.mean((t - m) * (t - m), axis=0, keepdims=True)
    h = g_ref[...] * (t - m) * lax.rsqrt(v + 1e-5) + o_ref[...]
    wei = jax.nn.sigmoid(h)
    out_ref[...] = 2.0 * x_ref[...] * wei + 2.0 * y_ref[...] * (1.0 - wei)


def _make_dense_mid(fin, fout):
    return pl.pallas_call(
        _dense_mid_body,
        out_shape=jax.ShapeDtypeStruct((N, fout), jnp.float32),
    )


def _make_dense_gate(fin, fout):
    return pl.pallas_call(
        _dense_gate_body,
        out_shape=jax.ShapeDtypeStruct((N, fout), jnp.float32),
    )


_dense1 = _make_dense_mid(64, 64)
_dense2 = _make_dense_gate(64, 32)
_dense3 = _make_dense_mid(32, 64)
_dense4 = _make_dense_gate(64, 32)


def kernel(x, y, senders, receivers, rel_pos, window_support, a,
           S1, W1, b1, g1, o1, S2, W2, b2, g2, o2,
           S3, W3, b3, g3, o3, S4, W4, b4, g4, o4):
    f32 = jnp.float32
    npad = EPAD - E
    pad_i = (jnp.arange(npad, dtype=jnp.int32) % N)
    send2 = jnp.concatenate([senders.astype(jnp.int32), pad_i]).reshape(ROWS, LANES)
    recv2 = jnp.concatenate([receivers.astype(jnp.int32), pad_i]).reshape(ROWS, LANES)
    zpad = jnp.zeros((npad,), f32)
    rpx = jnp.concatenate([rel_pos[:, 0].astype(f32), zpad]).reshape(ROWS, LANES)
    rpy = jnp.concatenate([rel_pos[:, 1].astype(f32), zpad]).reshape(ROWS, LANES)
    win = jnp.concatenate([window_support.astype(f32), zpad]).reshape(ROWS, LANES)
    sa = jnp.concatenate(
        [_corner_pack(S) for S in (S1, S2, S3, S4)]).astype(f32)

    w1, w2, w3, w4 = _edge_w(rpx, rpy, win, sa)

    a2 = a.astype(f32).reshape(N, 1)
    xa = jnp.concatenate([x, y], axis=1).astype(f32)

    conv64 = _make_conv(64)
    conv32 = _make_conv(32)
    p1 = conv64(xa, send2, recv2, w1)
    h1 = _dense1(p1, a2, W1, b1.reshape(1, -1), g1.reshape(1, -1),
                 o1.reshape(1, -1))
    p2 = conv64(h1, send2, recv2, w2)
    xo = _dense2(p2, a2, W2, b2.reshape(1, -1), g2.reshape(1, -1),
                 o2.reshape(1, -1), x, y)
    p3 = conv32(xo, send2, recv2, w3)
    h3 = _dense3(p3, a2, W3, b3.reshape(1, -1), g3.reshape(1, -1),
                 o3.reshape(1, -1))
    p4 = conv64(h3, send2, recv2, w4)
    return _dense4(p4, a2, W4, b4.reshape(1, -1), g4.reshape(1, -1),
                   o4.reshape(1, -1), x, y)


# speedup vs baseline: 138.6748x; 1.1281x over previous
import functools

import jax
import jax.numpy as jnp
from jax import lax
from jax.experimental import pallas as pl
from jax.experimental.pallas import tpu as pltpu
from jax.experimental.pallas import tpu_sc as plsc

N = 10000
E = 640000
LANES = 128
NC, NS = 2, 16
NW = NC * NS
RCH = 8
NCH = 20
RPW = RCH * NCH
ROWS = NW * RPW
EPAD = ROWS * LANES
ACCN = 10240
NPS = ACCN // NS
CH_E = RCH * LANES


def _edge_w_body(rpx_ref, rpy_ref, win_ref, sa_ref, o1_ref, o2_ref, o3_ref,
                 o4_ref):
    rx = rpx_ref[...]
    ry = rpy_ref[...]
    win = win_ref[...]
    ua = (jnp.clip(rx, -1.0, 1.0) + 1.0) * 1.5
    ub = (jnp.clip(ry, -1.0, 1.0) + 1.0) * 1.5
    ia = jnp.clip(jnp.floor(ua), 0.0, 2.0)
    ib = jnp.clip(jnp.floor(ub), 0.0, 2.0)
    fa = ua - ia
    fb = ub - ib
    c00 = (1.0 - fa) * (1.0 - fb)
    c01 = (1.0 - fa) * fb
    c10 = fa * (1.0 - fb)
    c11 = fa * fb
    ea = [(ia == float(k)).astype(jnp.float32) for k in range(3)]
    eb = [(ib == float(m)).astype(jnp.float32) for m in range(3)]
    outs = [o1_ref, o2_ref, o3_ref, o4_ref]
    for l in range(4):
        acc = None
        for k in range(3):
            for m in range(3):
                j = k * 3 + m
                br = (c00 * sa_ref[l * 36 + j]
                      + c01 * sa_ref[l * 36 + 9 + j]
                      + c10 * sa_ref[l * 36 + 18 + j]
                      + c11 * sa_ref[l * 36 + 27 + j])
                term = (ea[k] * eb[m]) * br
                acc = term if acc is None else acc + term
        outs[l][...] = acc * win


_BR = 64

_edge_w = pl.pallas_call(
    _edge_w_body,
    grid=(ROWS // _BR,),
    in_specs=[pl.BlockSpec((_BR, LANES), lambda i: (i, 0))] * 3
    + [pl.BlockSpec(memory_space=pltpu.SMEM)],
    out_specs=[pl.BlockSpec((_BR, LANES), lambda i: (i, 0))] * 4,
    out_shape=[jax.ShapeDtypeStruct((ROWS, LANES), jnp.float32)] * 4,
)


def _corner_pack(S):
    return jnp.stack(
        [S[0:3, 0:3], S[0:3, 1:4], S[1:4, 0:3], S[1:4, 1:4]]).reshape(36)


@functools.lru_cache(maxsize=None)
def _make_conv(F):
    mesh = plsc.VectorSubcoreMesh(core_axis_name="c", subcore_axis_name="s",
                                  num_cores=NC, num_subcores=NS)

    HC = 4
    HCE = HC * LANES
    NPASS = 4 if F == 64 else 1
    PRW = RPW // NPASS
    NHC = PRW // HC

    def body(feat, send, recv, wgt, out, sidx, ridx, wv, rows, acc, gsem,
             ssem):
        cid = lax.axis_index("c")
        sid = lax.axis_index("s")
        wid = cid * NS + sid

        def zrow(r, carry):
            for q in range(F // 16):
                rows[0, r, pl.ds(q * 16, 16)] = jnp.zeros((16,), jnp.float32)
                rows[1, r, pl.ds(q * 16, 16)] = jnp.zeros((16,), jnp.float32)
            return carry

        lax.fori_loop(0, HCE, zrow, 0)
        pltpu.sync_copy(rows.at[0], acc.at[pl.ds(sid * NPS, HCE)])
        pltpu.sync_copy(rows.at[1, pl.ds(0, NPS - HCE)],
                        acc.at[pl.ds(sid * NPS + HCE, NPS - HCE)])
        plsc.subcore_barrier()

        def fire_gathers(hc):
            p = hc & 1
            for j in range(HC):
                pltpu.async_copy(feat.at[sidx.at[hc * HC + j]],
                                 rows.at[p, pl.ds(j * LANES, LANES)], gsem)

        def drain_gathers(hc):
            p = hc & 1
            for j in range(HC):
                pltpu.make_async_copy(
                    feat.at[sidx.at[hc * HC + j]],
                    rows.at[p, pl.ds(j * LANES, LANES)], gsem).wait()

        def fire_scatters(hc):
            p = hc & 1
            for j in range(HC):
                pltpu.async_copy(rows.at[p, pl.ds(j * LANES, LANES)],
                                 acc.at[ridx.at[hc * HC + j]], ssem, add=True)

        def drain_scatters(hc):
            p = hc & 1
            for j in range(HC):
                pltpu.make_async_copy(
                    rows.at[p, pl.ds(j * LANES, LANES)],
                    acc.at[ridx.at[hc * HC + j]], ssem).wait()

        def pass_body(pass_, pcarry):
            row0 = wid * RPW + pass_ * PRW
            pltpu.sync_copy(send.at[pl.ds(row0, PRW)], sidx)
            pltpu.sync_copy(recv.at[pl.ds(row0, PRW)], ridx)
            pltpu.sync_copy(wgt.at[pl.ds(row0, PRW)], wv)

            fire_gathers(jnp.int32(0))

            def step(hc, carry):
                p = hc & 1
                drain_gathers(hc)

                @pl.when(hc >= 1)
                def _():
                    drain_scatters(hc - 1)

                @pl.when(hc + 1 < NHC)
                def _():
                    fire_gathers(hc + 1)

                for j in range(HC):
                    @plsc.parallel_loop(0, LANES // 16, step=1, unroll=2)
                    def scale_grp(i, j=j):
                        w16 = wv[hc * HC + j, pl.ds(i * 16, 16)]
                        for l2 in range(16):
                            w_s = w16[l2]
                            e = j * LANES + i * 16 + l2
                            for q in range(F // 16):
                                rows[p, e, pl.ds(q * 16, 16)] = (
                                    rows[p, e, pl.ds(q * 16, 16)] * w_s)

                    pltpu.async_copy(rows.at[p, pl.ds(j * LANES, LANES)],
                                     acc.at[ridx.at[hc * HC + j]], ssem,
                                     add=True)
                return carry

            lax.fori_loop(0, NHC, step, 0)
            drain_scatters(jnp.int32(NHC - 1))
            return pcarry

        lax.fori_loop(0, NPASS, pass_body, 0)
        plsc.subcore_barrier()

        pltpu.sync_copy(acc.at[pl.ds(sid * NPS, HCE)], rows.at[0])
        pltpu.sync_copy(rows.at[0],
                        out.at[pl.ds(cid * ACCN + sid * NPS, HCE)])
        pltpu.sync_copy(acc.at[pl.ds(sid * NPS + HCE, NPS - HCE)],
                        rows.at[1, pl.ds(0, NPS - HCE)])
        pltpu.sync_copy(rows.at[1, pl.ds(0, NPS - HCE)],
                        out.at[pl.ds(cid * ACCN + sid * NPS + HCE,
                                     NPS - HCE)])

    return pl.kernel(
        body,
        out_type=jax.ShapeDtypeStruct((NC * ACCN, F), jnp.float32),
        mesh=mesh,
        compiler_params=pltpu.CompilerParams(use_tc_tiling_on_sc=False),
        scratch_types=[
            pltpu.VMEM((PRW, LANES), jnp.int32),
            pltpu.VMEM((PRW, LANES), jnp.int32),
            pltpu.VMEM((PRW, LANES), jnp.float32),
            pltpu.VMEM((2, HCE, F), jnp.float32),
            pltpu.VMEM_SHARED((ACCN, F), jnp.float32),
            pltpu.SemaphoreType.DMA,
            pltpu.SemaphoreType.DMA,
        ],
    )


def _dense_mid_body(p_ref, a_ref, w_ref, b_ref, g_ref, o_ref, out_ref):
    agg = (p_ref[0:N, :] + p_ref[ACCN:ACCN + N, :]) / a_ref[...]
    t = jnp.dot(agg, w_ref[...], preferred_element_type=jnp.float32) + b_ref[...]
    m = jnp.mean(t, axis=0, keepdims=True)
    v = jnp.mean((t - m) * (t - m), axis=0, keepdims=True)
    h = g_ref[...] * (t - m) * lax.rsqrt(v + 1e-5) + o_ref[...]
    out_ref[...] = jnp.maximum(h, 0.0)


def _dense_gate_body(p_ref, a_ref, w_ref, b_ref, g_ref, o_ref, x_ref, y_ref,
                     out_ref):
    agg = (p_ref[0:N, :] + p_ref[ACCN:ACCN + N, :]) / a_ref[...]
    t = jnp.dot(agg, w_ref[...], preferred_element_type=jnp.float32) + b_ref[...]
    m = jnp.mean(t, axis=0, keepdims=True)
    v = jnp.mean((t - m) * (t - m), axis=0, keepdims=True)
    h = g_ref[...] * (t - m) * lax.rsqrt(v + 1e-5) + o_ref[...]
    wei = jax.nn.sigmoid(h)
    out_ref[...] = 2.0 * x_ref[...] * wei + 2.0 * y_ref[...] * (1.0 - wei)


def _make_dense_mid(fin, fout):
    return pl.pallas_call(
        _dense_mid_body,
        out_shape=jax.ShapeDtypeStruct((N, fout), jnp.float32),
    )


def _make_dense_gate(fin, fout):
    return pl.pallas_call(
        _dense_gate_body,
        out_shape=jax.ShapeDtypeStruct((N, fout), jnp.float32),
    )


_dense1 = _make_dense_mid(64, 64)
_dense2 = _make_dense_gate(64, 32)
_dense3 = _make_dense_mid(32, 64)
_dense4 = _make_dense_gate(64, 32)


def kernel(x, y, senders, receivers, rel_pos, window_support, a,
           S1, W1, b1, g1, o1, S2, W2, b2, g2, o2,
           S3, W3, b3, g3, o3, S4, W4, b4, g4, o4):
    f32 = jnp.float32
    npad = EPAD - E
    pad_i = (jnp.arange(npad, dtype=jnp.int32) % N)
    send2 = jnp.concatenate([senders.astype(jnp.int32), pad_i]).reshape(ROWS, LANES)
    recv2 = jnp.concatenate([receivers.astype(jnp.int32), pad_i]).reshape(ROWS, LANES)
    zpad = jnp.zeros((npad,), f32)
    rpx = jnp.concatenate([rel_pos[:, 0].astype(f32), zpad]).reshape(ROWS, LANES)
    rpy = jnp.concatenate([rel_pos[:, 1].astype(f32), zpad]).reshape(ROWS, LANES)
    win = jnp.concatenate([window_support.astype(f32), zpad]).reshape(ROWS, LANES)
    sa = jnp.concatenate(
        [_corner_pack(S) for S in (S1, S2, S3, S4)]).astype(f32)

    w1, w2, w3, w4 = _edge_w(rpx, rpy, win, sa)

    a2 = a.astype(f32).reshape(N, 1)
    xa = jnp.concatenate([x, y], axis=1).astype(f32)

    conv64 = _make_conv(64)
    conv32 = _make_conv(32)
    p1 = conv64(xa, send2, recv2, w1)
    h1 = _dense1(p1, a2, W1, b1.reshape(1, -1), g1.reshape(1, -1),
                 o1.reshape(1, -1))
    p2 = conv64(h1, send2, recv2, w2)
    xo = _dense2(p2, a2, W2, b2.reshape(1, -1), g2.reshape(1, -1),
                 o2.reshape(1, -1), x, y)
    p3 = conv32(xo, send2, recv2, w3)
    h3 = _dense3(p3, a2, W3, b3.reshape(1, -1), g3.reshape(1, -1),
                 o3.reshape(1, -1))
    p4 = conv64(h3, send2, recv2, w4)
    return _dense4(p4, a2, W4, b4.reshape(1, -1), g4.reshape(1, -1),
                   o4.reshape(1, -1), x, y)


# speedup vs baseline: 152.2895x; 1.0982x over previous
import functools

import jax
import jax.numpy as jnp
from jax import lax
from jax.experimental import pallas as pl
from jax.experimental.pallas import tpu as pltpu
from jax.experimental.pallas import tpu_sc as plsc

N = 10000
E = 640000
LANES = 128
NC, NS = 2, 16
NW = NC * NS
RCH = 8
NCH = 20
RPW = RCH * NCH
ROWS = NW * RPW
EPAD = ROWS * LANES
ACCN = 10240
NPS = ACCN // NS
CH_E = RCH * LANES


def _edge_w_body(rpx_ref, rpy_ref, win_ref, sa_ref, o1_ref, o2_ref, o3_ref,
                 o4_ref):
    rx = rpx_ref[...]
    ry = rpy_ref[...]
    win = win_ref[...]
    ua = (jnp.clip(rx, -1.0, 1.0) + 1.0) * 1.5
    ub = (jnp.clip(ry, -1.0, 1.0) + 1.0) * 1.5
    ia = jnp.clip(jnp.floor(ua), 0.0, 2.0)
    ib = jnp.clip(jnp.floor(ub), 0.0, 2.0)
    fa = ua - ia
    fb = ub - ib
    c00 = (1.0 - fa) * (1.0 - fb)
    c01 = (1.0 - fa) * fb
    c10 = fa * (1.0 - fb)
    c11 = fa * fb
    ea = [(ia == float(k)).astype(jnp.float32) for k in range(3)]
    eb = [(ib == float(m)).astype(jnp.float32) for m in range(3)]
    outs = [o1_ref, o2_ref, o3_ref, o4_ref]
    for l in range(4):
        acc = None
        for k in range(3):
            for m in range(3):
                j = k * 3 + m
                br = (c00 * sa_ref[l * 36 + j]
                      + c01 * sa_ref[l * 36 + 9 + j]
                      + c10 * sa_ref[l * 36 + 18 + j]
                      + c11 * sa_ref[l * 36 + 27 + j])
                term = (ea[k] * eb[m]) * br
                acc = term if acc is None else acc + term
        outs[l][...] = acc * win


_BR = 64

_edge_w = pl.pallas_call(
    _edge_w_body,
    grid=(ROWS // _BR,),
    in_specs=[pl.BlockSpec((_BR, LANES), lambda i: (i, 0))] * 3
    + [pl.BlockSpec(memory_space=pltpu.SMEM)],
    out_specs=[pl.BlockSpec((_BR, LANES), lambda i: (i, 0))] * 4,
    out_shape=[jax.ShapeDtypeStruct((ROWS, LANES), jnp.float32)] * 4,
)


def _corner_pack(S):
    return jnp.stack(
        [S[0:3, 0:3], S[0:3, 1:4], S[1:4, 0:3], S[1:4, 1:4]]).reshape(36)


@functools.lru_cache(maxsize=None)
def _make_conv(F):
    mesh = plsc.VectorSubcoreMesh(core_axis_name="c", subcore_axis_name="s",
                                  num_cores=NC, num_subcores=NS)

    HC = 4
    HCE = HC * LANES
    NPASS = 4 if F == 64 else 1
    PRW = RPW // NPASS
    NHC = PRW // HC

    def body(feat, send, recv, wgt, out, sidx, ridx, wv, rows, acc, gsem,
             ssem):
        cid = lax.axis_index("c")
        sid = lax.axis_index("s")
        wid = cid * NS + sid

        def zrow(r, carry):
            for q in range(F // 16):
                rows[0, r, pl.ds(q * 16, 16)] = jnp.zeros((16,), jnp.float32)
                rows[1, r, pl.ds(q * 16, 16)] = jnp.zeros((16,), jnp.float32)
            return carry

        lax.fori_loop(0, HCE, zrow, 0)
        pltpu.sync_copy(rows.at[0], acc.at[pl.ds(sid * NPS, HCE)])
        pltpu.sync_copy(rows.at[1, pl.ds(0, NPS - HCE)],
                        acc.at[pl.ds(sid * NPS + HCE, NPS - HCE)])
        plsc.subcore_barrier()

        def fire_gathers(hc):
            p = hc & 1
            for j in range(HC):
                pltpu.async_copy(feat.at[sidx.at[hc * HC + j]],
                                 rows.at[p, pl.ds(j * LANES, LANES)], gsem)

        def drain_gathers(hc):
            p = hc & 1
            for j in range(HC):
                pltpu.make_async_copy(
                    feat.at[sidx.at[hc * HC + j]],
                    rows.at[p, pl.ds(j * LANES, LANES)], gsem).wait()

        def fire_scatters(hc):
            p = hc & 1
            for j in range(HC):
                pltpu.async_copy(rows.at[p, pl.ds(j * LANES, LANES)],
                                 acc.at[ridx.at[hc * HC + j]], ssem, add=True)

        def drain_scatters(hc):
            p = hc & 1
            for j in range(HC):
                pltpu.make_async_copy(
                    rows.at[p, pl.ds(j * LANES, LANES)],
                    acc.at[ridx.at[hc * HC + j]], ssem).wait()

        def pass_body(pass_, pcarry):
            row0 = wid * RPW + pass_ * PRW
            pltpu.sync_copy(send.at[pl.ds(row0, PRW)], sidx)
            pltpu.sync_copy(recv.at[pl.ds(row0, PRW)], ridx)
            pltpu.sync_copy(wgt.at[pl.ds(row0, PRW)], wv)

            fire_gathers(jnp.int32(0))

            def step(hc, carry):
                p = hc & 1
                drain_gathers(hc)

                @pl.when(hc >= 1)
                def _():
                    drain_scatters(hc - 1)

                @pl.when(hc + 1 < NHC)
                def _():
                    fire_gathers(hc + 1)

                for j in range(HC):
                    @plsc.parallel_loop(0, LANES // 16, step=1, unroll=4)
                    def scale_grp(i, j=j):
                        w16 = wv[hc * HC + j, pl.ds(i * 16, 16)]
                        for l2 in range(16):
                            w_s = w16[l2]
                            e = j * LANES + i * 16 + l2
                            for q in range(F // 16):
                                rows[p, e, pl.ds(q * 16, 16)] = (
                                    rows[p, e, pl.ds(q * 16, 16)] * w_s)

                    pltpu.async_copy(rows.at[p, pl.ds(j * LANES, LANES)],
                                     acc.at[ridx.at[hc * HC + j]], ssem,
                                     add=True)
                return carry

            lax.fori_loop(0, NHC, step, 0)
            drain_scatters(jnp.int32(NHC - 1))
            return pcarry

        lax.fori_loop(0, NPASS, pass_body, 0)
        plsc.subcore_barrier()

        pltpu.sync_copy(acc.at[pl.ds(sid * NPS, HCE)], rows.at[0])
        pltpu.sync_copy(rows.at[0],
                        out.at[pl.ds(cid * ACCN + sid * NPS, HCE)])
        pltpu.sync_copy(acc.at[pl.ds(sid * NPS + HCE, NPS - HCE)],
                        rows.at[1, pl.ds(0, NPS - HCE)])
        pltpu.sync_copy(rows.at[1, pl.ds(0, NPS - HCE)],
                        out.at[pl.ds(cid * ACCN + sid * NPS + HCE,
                                     NPS - HCE)])

    return pl.kernel(
        body,
        out_type=jax.ShapeDtypeStruct((NC * ACCN, F), jnp.float32),
        mesh=mesh,
        compiler_params=pltpu.CompilerParams(use_tc_tiling_on_sc=False),
        scratch_types=[
            pltpu.VMEM((PRW, LANES), jnp.int32),
            pltpu.VMEM((PRW, LANES), jnp.int32),
            pltpu.VMEM((PRW, LANES), jnp.float32),
            pltpu.VMEM((2, HCE, F), jnp.float32),
            pltpu.VMEM_SHARED((ACCN, F), jnp.float32),
            pltpu.SemaphoreType.DMA,
            pltpu.SemaphoreType.DMA,
        ],
    )


def _dense_mid_body(p_ref, a_ref, w_ref, b_ref, g_ref, o_ref, out_ref):
    agg = (p_ref[0:N, :] + p_ref[ACCN:ACCN + N, :]) / a_ref[...]
    t = jnp.dot(agg, w_ref[...], preferred_element_type=jnp.float32) + b_ref[...]
    m = jnp.mean(t, axis=0, keepdims=True)
    v = jnp.mean((t - m) * (t - m), axis=0, keepdims=True)
    h = g_ref[...] * (t - m) * lax.rsqrt(v + 1e-5) + o_ref[...]
    out_ref[...] = jnp.maximum(h, 0.0)


def _dense_gate_body(p_ref, a_ref, w_ref, b_ref, g_ref, o_ref, x_ref, y_ref,
                     out_ref):
    agg = (p_ref[0:N, :] + p_ref[ACCN:ACCN + N, :]) / a_ref[...]
    t = jnp.dot(agg, w_ref[...], preferred_element_type=jnp.float32) + b_ref[...]
    m = jnp.mean(t, axis=0, keepdims=True)
    v = jnp.mean((t - m) * (t - m), axis=0, keepdims=True)
    h = g_ref[...] * (t - m) * lax.rsqrt(v + 1e-5) + o_ref[...]
    wei = jax.nn.sigmoid(h)
    out_ref[...] = 2.0 * x_ref[...] * wei + 2.0 * y_ref[...] * (1.0 - wei)


def _make_dense_mid(fin, fout):
    return pl.pallas_call(
        _dense_mid_body,
        out_shape=jax.ShapeDtypeStruct((N, fout), jnp.float32),
    )


def _make_dense_gate(fin, fout):
    return pl.pallas_call(
        _dense_gate_body,
        out_shape=jax.ShapeDtypeStruct((N, fout), jnp.float32),
    )


_dense1 = _make_dense_mid(64, 64)
_dense2 = _make_dense_gate(64, 32)
_dense3 = _make_dense_mid(32, 64)
_dense4 = _make_dense_gate(64, 32)


def kernel(x, y, senders, receivers, rel_pos, window_support, a,
           S1, W1, b1, g1, o1, S2, W2, b2, g2, o2,
           S3, W3, b3, g3, o3, S4, W4, b4, g4, o4):
    f32 = jnp.float32
    npad = EPAD - E
    pad_i = (jnp.arange(npad, dtype=jnp.int32) % N)
    send2 = jnp.concatenate([senders.astype(jnp.int32), pad_i]).reshape(ROWS, LANES)
    recv2 = jnp.concatenate([receivers.astype(jnp.int32), pad_i]).reshape(ROWS, LANES)
    zpad = jnp.zeros((npad,), f32)
    rpx = jnp.concatenate([rel_pos[:, 0].astype(f32), zpad]).reshape(ROWS, LANES)
    rpy = jnp.concatenate([rel_pos[:, 1].astype(f32), zpad]).reshape(ROWS, LANES)
    win = jnp.concatenate([window_support.astype(f32), zpad]).reshape(ROWS, LANES)
    sa = jnp.concatenate(
        [_corner_pack(S) for S in (S1, S2, S3, S4)]).astype(f32)

    w1, w2, w3, w4 = _edge_w(rpx, rpy, win, sa)

    a2 = a.astype(f32).reshape(N, 1)
    xa = jnp.concatenate([x, y], axis=1).astype(f32)

    conv64 = _make_conv(64)
    conv32 = _make_conv(32)
    p1 = conv64(xa, send2, recv2, w1)
    h1 = _dense1(p1, a2, W1, b1.reshape(1, -1), g1.reshape(1, -1),
                 o1.reshape(1, -1))
    p2 = conv64(h1, send2, recv2, w2)
    xo = _dense2(p2, a2, W2, b2.reshape(1, -1), g2.reshape(1, -1),
                 o2.reshape(1, -1), x, y)
    p3 = conv32(xo, send2, recv2, w3)
    h3 = _dense3(p3, a2, W3, b3.reshape(1, -1), g3.reshape(1, -1),
                 o3.reshape(1, -1))
    p4 = conv64(h3, send2, recv2, w4)
    return _dense4(p4, a2, W4, b4.reshape(1, -1), g4.reshape(1, -1),
                   o4.reshape(1, -1), x, y)


# speedup vs baseline: 159.7094x; 1.0487x over previous
import functools

import jax
import jax.numpy as jnp
from jax import lax
from jax.experimental import pallas as pl
from jax.experimental.pallas import tpu as pltpu
from jax.experimental.pallas import tpu_sc as plsc

N = 10000
E = 640000
LANES = 128
NC, NS = 2, 16
NW = NC * NS
RCH = 8
NCH = 20
RPW = RCH * NCH
ROWS = NW * RPW
EPAD = ROWS * LANES
ACCN = 10240
NPS = ACCN // NS
CH_E = RCH * LANES


def _edge_w_body(rpx_ref, rpy_ref, win_ref, sa_ref, o1_ref, o2_ref, o3_ref,
                 o4_ref):
    rx = rpx_ref[...]
    ry = rpy_ref[...]
    win = win_ref[...]
    ua = (jnp.clip(rx, -1.0, 1.0) + 1.0) * 1.5
    ub = (jnp.clip(ry, -1.0, 1.0) + 1.0) * 1.5
    ia = jnp.clip(jnp.floor(ua), 0.0, 2.0)
    ib = jnp.clip(jnp.floor(ub), 0.0, 2.0)
    fa = ua - ia
    fb = ub - ib
    c00 = (1.0 - fa) * (1.0 - fb)
    c01 = (1.0 - fa) * fb
    c10 = fa * (1.0 - fb)
    c11 = fa * fb
    ea = [(ia == float(k)).astype(jnp.float32) for k in range(3)]
    eb = [(ib == float(m)).astype(jnp.float32) for m in range(3)]
    outs = [o1_ref, o2_ref, o3_ref, o4_ref]
    for l in range(4):
        acc = None
        for k in range(3):
            for m in range(3):
                j = k * 3 + m
                br = (c00 * sa_ref[l * 36 + j]
                      + c01 * sa_ref[l * 36 + 9 + j]
                      + c10 * sa_ref[l * 36 + 18 + j]
                      + c11 * sa_ref[l * 36 + 27 + j])
                term = (ea[k] * eb[m]) * br
                acc = term if acc is None else acc + term
        outs[l][...] = acc * win


_BR = 64

_edge_w = pl.pallas_call(
    _edge_w_body,
    grid=(ROWS // _BR,),
    in_specs=[pl.BlockSpec((_BR, LANES), lambda i: (i, 0))] * 3
    + [pl.BlockSpec(memory_space=pltpu.SMEM)],
    out_specs=[pl.BlockSpec((_BR, LANES), lambda i: (i, 0))] * 4,
    out_shape=[jax.ShapeDtypeStruct((ROWS, LANES), jnp.float32)] * 4,
)


def _corner_pack(S):
    return jnp.stack(
        [S[0:3, 0:3], S[0:3, 1:4], S[1:4, 0:3], S[1:4, 1:4]]).reshape(36)


@functools.lru_cache(maxsize=None)
def _make_conv(F):
    mesh = plsc.VectorSubcoreMesh(core_axis_name="c", subcore_axis_name="s",
                                  num_cores=NC, num_subcores=NS)

    HC = 4
    HCE = HC * LANES
    NPASS = 4 if F == 64 else 1
    PRW = RPW // NPASS
    NHC = PRW // HC

    def body(feat, send, recv, wgt, out, sidx, ridx, wv, rows, acc, gsem,
             ssem):
        cid = lax.axis_index("c")
        sid = lax.axis_index("s")
        wid = cid * NS + sid

        def zrow(r, carry):
            for q in range(F // 16):
                rows[0, r, pl.ds(q * 16, 16)] = jnp.zeros((16,), jnp.float32)
                rows[1, r, pl.ds(q * 16, 16)] = jnp.zeros((16,), jnp.float32)
            return carry

        lax.fori_loop(0, HCE, zrow, 0)
        pltpu.sync_copy(rows.at[0], acc.at[pl.ds(sid * NPS, HCE)])
        pltpu.sync_copy(rows.at[1, pl.ds(0, NPS - HCE)],
                        acc.at[pl.ds(sid * NPS + HCE, NPS - HCE)])
        plsc.subcore_barrier()

        def fire_gathers(hc):
            p = hc & 1
            for j in range(HC):
                pltpu.async_copy(feat.at[sidx.at[hc * HC + j]],
                                 rows.at[p, pl.ds(j * LANES, LANES)], gsem)

        def drain_gathers(hc):
            p = hc & 1
            for j in range(HC):
                pltpu.make_async_copy(
                    feat.at[sidx.at[hc * HC + j]],
                    rows.at[p, pl.ds(j * LANES, LANES)], gsem).wait()

        def fire_scatters(hc):
            p = hc & 1
            for j in range(HC):
                pltpu.async_copy(rows.at[p, pl.ds(j * LANES, LANES)],
                                 acc.at[ridx.at[hc * HC + j]], ssem, add=True)

        def drain_scatters(hc):
            p = hc & 1
            for j in range(HC):
                pltpu.make_async_copy(
                    rows.at[p, pl.ds(j * LANES, LANES)],
                    acc.at[ridx.at[hc * HC + j]], ssem).wait()

        def pass_body(pass_, pcarry):
            row0 = wid * RPW + pass_ * PRW
            pltpu.sync_copy(send.at[pl.ds(row0, PRW)], sidx)
            pltpu.sync_copy(recv.at[pl.ds(row0, PRW)], ridx)
            pltpu.sync_copy(wgt.at[pl.ds(row0, PRW)], wv)

            fire_gathers(jnp.int32(0))

            def step(hc, carry):
                p = hc & 1
                drain_gathers(hc)

                @pl.when(hc >= 1)
                def _():
                    drain_scatters(hc - 1)

                @pl.when(hc + 1 < NHC)
                def _():
                    fire_gathers(hc + 1)

                for j in range(HC):
                    @plsc.parallel_loop(0, LANES // 16, step=1, unroll=8)
                    def scale_grp(i, j=j):
                        w16 = wv[hc * HC + j, pl.ds(i * 16, 16)]
                        for l2 in range(16):
                            w_s = w16[l2]
                            e = j * LANES + i * 16 + l2
                            for q in range(F // 16):
                                rows[p, e, pl.ds(q * 16, 16)] = (
                                    rows[p, e, pl.ds(q * 16, 16)] * w_s)

                    pltpu.async_copy(rows.at[p, pl.ds(j * LANES, LANES)],
                                     acc.at[ridx.at[hc * HC + j]], ssem,
                                     add=True)
                return carry

            lax.fori_loop(0, NHC, step, 0)
            drain_scatters(jnp.int32(NHC - 1))
            return pcarry

        lax.fori_loop(0, NPASS, pass_body, 0)
        plsc.subcore_barrier()

        pltpu.sync_copy(acc.at[pl.ds(sid * NPS, HCE)], rows.at[0])
        pltpu.sync_copy(rows.at[0],
                        out.at[pl.ds(cid * ACCN + sid * NPS, HCE)])
        pltpu.sync_copy(acc.at[pl.ds(sid * NPS + HCE, NPS - HCE)],
                        rows.at[1, pl.ds(0, NPS - HCE)])
        pltpu.sync_copy(rows.at[1, pl.ds(0, NPS - HCE)],
                        out.at[pl.ds(cid * ACCN + sid * NPS + HCE,
                                     NPS - HCE)])

    return pl.kernel(
        body,
        out_type=jax.ShapeDtypeStruct((NC * ACCN, F), jnp.float32),
        mesh=mesh,
        compiler_params=pltpu.CompilerParams(use_tc_tiling_on_sc=False),
        scratch_types=[
            pltpu.VMEM((PRW, LANES), jnp.int32),
            pltpu.VMEM((PRW, LANES), jnp.int32),
            pltpu.VMEM((PRW, LANES), jnp.float32),
            pltpu.VMEM((2, HCE, F), jnp.float32),
            pltpu.VMEM_SHARED((ACCN, F), jnp.float32),
            pltpu.SemaphoreType.DMA,
            pltpu.SemaphoreType.DMA,
        ],
    )


def _dense_mid_body(p_ref, a_ref, w_ref, b_ref, g_ref, o_ref, out_ref):
    agg = (p_ref[0:N, :] + p_ref[ACCN:ACCN + N, :]) / a_ref[...]
    t = jnp.dot(agg, w_ref[...], preferred_element_type=jnp.float32) + b_ref[...]
    m = jnp.mean(t, axis=0, keepdims=True)
    v = jnp.mean((t - m) * (t - m), axis=0, keepdims=True)
    h = g_ref[...] * (t - m) * lax.rsqrt(v + 1e-5) + o_ref[...]
    out_ref[...] = jnp.maximum(h, 0.0)


def _dense_gate_body(p_ref, a_ref, w_ref, b_ref, g_ref, o_ref, x_ref, y_ref,
                     out_ref):
    agg = (p_ref[0:N, :] + p_ref[ACCN:ACCN + N, :]) / a_ref[...]
    t = jnp.dot(agg, w_ref[...], preferred_element_type=jnp.float32) + b_ref[...]
    m = jnp.mean(t, axis=0, keepdims=True)
    v = jnp.mean((t - m) * (t - m), axis=0, keepdims=True)
    h = g_ref[...] * (t - m) * lax.rsqrt(v + 1e-5) + o_ref[...]
    wei = jax.nn.sigmoid(h)
    out_ref[...] = 2.0 * x_ref[...] * wei + 2.0 * y_ref[...] * (1.0 - wei)


def _make_dense_mid(fin, fout):
    return pl.pallas_call(
        _dense_mid_body,
        out_shape=jax.ShapeDtypeStruct((N, fout), jnp.float32),
    )


def _make_dense_gate(fin, fout):
    return pl.pallas_call(
        _dense_gate_body,
        out_shape=jax.ShapeDtypeStruct((N, fout), jnp.float32),
    )


_dense1 = _make_dense_mid(64, 64)
_dense2 = _make_dense_gate(64, 32)
_dense3 = _make_dense_mid(32, 64)
_dense4 = _make_dense_gate(64, 32)


def kernel(x, y, senders, receivers, rel_pos, window_support, a,
           S1, W1, b1, g1, o1, S2, W2, b2, g2, o2,
           S3, W3, b3, g3, o3, S4, W4, b4, g4, o4):
    f32 = jnp.float32
    npad = EPAD - E
    pad_i = (jnp.arange(npad, dtype=jnp.int32) % N)
    send2 = jnp.concatenate([senders.astype(jnp.int32), pad_i]).reshape(ROWS, LANES)
    recv2 = jnp.concatenate([receivers.astype(jnp.int32), pad_i]).reshape(ROWS, LANES)
    zpad = jnp.zeros((npad,), f32)
    rpx = jnp.concatenate([rel_pos[:, 0].astype(f32), zpad]).reshape(ROWS, LANES)
    rpy = jnp.concatenate([rel_pos[:, 1].astype(f32), zpad]).reshape(ROWS, LANES)
    win = jnp.concatenate([window_support.astype(f32), zpad]).reshape(ROWS, LANES)
    sa = jnp.concatenate(
        [_corner_pack(S) for S in (S1, S2, S3, S4)]).astype(f32)

    w1, w2, w3, w4 = _edge_w(rpx, rpy, win, sa)

    a2 = a.astype(f32).reshape(N, 1)
    xa = jnp.concatenate([x, y], axis=1).astype(f32)

    conv64 = _make_conv(64)
    conv32 = _make_conv(32)
    p1 = conv64(xa, send2, recv2, w1)
    h1 = _dense1(p1, a2, W1, b1.reshape(1, -1), g1.reshape(1, -1),
                 o1.reshape(1, -1))
    p2 = conv64(h1, send2, recv2, w2)
    xo = _dense2(p2, a2, W2, b2.reshape(1, -1), g2.reshape(1, -1),
                 o2.reshape(1, -1), x, y)
    p3 = conv32(xo, send2, recv2, w3)
    h3 = _dense3(p3, a2, W3, b3.reshape(1, -1), g3.reshape(1, -1),
                 o3.reshape(1, -1))
    p4 = conv64(h3, send2, recv2, w4)
    return _dense4(p4, a2, W4, b4.reshape(1, -1), g4.reshape(1, -1),
                   o4.reshape(1, -1), x, y)
